# baseline (speedup 1.0000x reference)
import functools

import jax
import jax.numpy as jnp
from jax import lax
from jax.experimental import pallas as pl
from jax.experimental.pallas import tpu as pltpu
from jax.experimental.pallas import tpu_sc as plsc

_B = 16384
_D = 64
_NC = 2
_NS = 16
_L = 16
_NBLK = 7813
_SPT = 489
_G = 4
_NG = (_SPT + _G - 1) // _G
_CAP = 1536
_GCAP = 40
_SROWS = 64
_OUTROWS = _B + _SROWS
_DUMP = _B
_BPT = _B // _NS
_XCH = 2048

_mesh = plsc.VectorSubcoreMesh(core_axis_name="c", subcore_axis_name="s",
                               num_cores=_NC, num_subcores=_NS)


@functools.partial(
    pl.kernel,
    out_type=(
        jax.ShapeDtypeStruct((_OUTROWS, 128), jnp.float32),
        jax.ShapeDtypeStruct((_OUTROWS, 128), jnp.float32),
        jax.ShapeDtypeStruct((_B,), jnp.float32),
        jax.ShapeDtypeStruct((_B,), jnp.float32),
    ),
    mesh=_mesh,
    compiler_params=pltpu.CompilerParams(
        needs_layout_passes=False,
        use_tc_tiling_on_sc=True,
        disable_bounds_checks=True,
    ),
    scratch_types=[
        pltpu.VMEM((2 * _XCH,), jnp.int32),
        pltpu.VMEM((_CAP,), jnp.int32),
        pltpu.VMEM((_CAP,), jnp.int32),
        pltpu.VMEM((_NG * _GCAP,), jnp.int32),
        pltpu.VMEM((_NG * _GCAP,), jnp.int32),
        pltpu.VMEM((_D, _G * 128), jnp.float32),
        pltpu.VMEM((_D, _G * 128), jnp.float32),
        pltpu.VMEM((_D, _G * 128), jnp.float32),
        pltpu.VMEM((_SROWS, 128), jnp.float32),
        pltpu.VMEM((_SROWS,), jnp.int32),
        pltpu.VMEM((_BPT,), jnp.int32),
        pltpu.VMEM((_BPT,), jnp.float32),
        pltpu.SMEM((_SPT + 1,), jnp.int32),
        pltpu.SMEM((_SPT + 1,), jnp.int32),
        pltpu.SemaphoreType.DMA,
        pltpu.SemaphoreType.DMA,
        pltpu.SemaphoreType.DMA,
        pltpu.SemaphoreType.DMA,
        pltpu.SemaphoreType.DMA,
    ],
)
def _gather_phase(x_hbm, ut_hbm, it_hbm, ub_hbm, ib_hbm,
                  urows_hbm, irows_hbm, ubg_hbm, ibg_hbm,
                  xc_v, fb_v, fr_v, sb_v, sr_v, gbuf0_v, gbuf1_v, gbuf2_v,
                  stage_v, pos_v, bidx_v, bval_v,
                  bin_s, end_s, sem0, sem1, sem2, sem_sc, sem_b):
    cid = lax.axis_index("c")
    sid = lax.axis_index("s")
    lanes = lax.iota(jnp.int32, _L)
    lane0 = lanes == 0
    tau = cid

    tstart = sid * _SPT
    tend = jnp.minimum(tstart + _SPT, _NBLK)
    nbins = tend - tstart

    bbase = sid * _BPT
    pltpu.sync_copy(x_hbm.at[pl.ds(2 * bbase, 2 * _BPT)],
                    xc_v.at[pl.ds(0, 2 * _BPT)])

    def bias_idx_body(k, _):
        bidx_v[pl.ds(k * _L, _L)] = plsc.load_gather(
            xc_v, [2 * (lanes + k * _L) + tau])
        return 0

    lax.fori_loop(0, _BPT // _L, bias_idx_body, 0, unroll=4)

    bufs = (gbuf0_v, gbuf1_v, gbuf2_v)
    sems = (sem0, sem1, sem2)

    def gclamp(g):
        gstart = tstart + _G * g
        return jnp.minimum(gstart, _NBLK - _G)

    def fire(g, buf, sem):
        gstart = tstart + _G * g
        off = pl.multiple_of(gclamp(g) * 128, 128)

        @pl.when((gstart < tend) & (cid == 0))
        def _():
            pltpu.async_copy(ut_hbm.at[:, pl.ds(off, _G * 128)], buf, sem)

        @pl.when((gstart < tend) & (cid == 1))
        def _():
            pltpu.async_copy(it_hbm.at[:, pl.ds(off, _G * 128)], buf, sem)

    fire(0, bufs[0], sems[0])
    fire(1, bufs[1], sems[1])
    fire(2, bufs[2], sems[2])

    @pl.when(cid == 0)
    def _():
        pltpu.async_copy(ub_hbm.at[bidx_v], bval_v, sem_b)

    @pl.when(cid == 1)
    def _():
        pltpu.async_copy(ib_hbm.at[bidx_v], bval_v, sem_b)

    def chunk_filt(c8, cnt0):
        pltpu.sync_copy(x_hbm.at[pl.ds(c8 * 2 * _XCH, 2 * _XCH)], xc_v)

        def filt_body(k, cnt):
            idx = plsc.load_gather(xc_v, [2 * (lanes + k * _L) + tau])
            slab = lax.shift_right_logical(idx, 7)
            m = (slab >= tstart) & (slab < tstart + _SPT)
            rank = plsc.cumsum(m.astype(jnp.int32))
            dest = cnt + rank - 1
            b = c8 * _XCH + k * _L + lanes
            plsc.store_scatter(fb_v, [dest], b, mask=m)
            plsc.store_scatter(fr_v, [dest], idx, mask=m)
            return cnt + plsc.all_reduce_population_count(m)[0]

        return lax.fori_loop(0, _XCH // _L, filt_body, cnt0, unroll=4)

    cnt = lax.fori_loop(0, _B // _XCH, chunk_filt, jnp.int32(0))

    def zero_body(s, _):
        bin_s[s] = 0
        return 0

    lax.fori_loop(0, _NG + 1, zero_body, 0)

    def place_body(e, _):
        ev = jnp.full((_L,), e, jnp.int32)
        idx = plsc.load_gather(fr_v, [ev])
        b = plsc.load_gather(fb_v, [ev])
        g = lax.shift_right_logical(
            lax.shift_right_logical(idx, 7)[0] - tstart, 2)
        o = jnp.minimum(bin_s[g], _GCAP - 1)
        bin_s[g] = o + 1
        ov = jnp.full((_L,), g * _GCAP + o, jnp.int32)
        plsc.store_scatter(sb_v, [ov], b, mask=lane0)
        plsc.store_scatter(sr_v, [ov], idx, mask=lane0)
        return 0

    lax.fori_loop(0, cnt, place_body, 0)

    def init_pos(k, _):
        pos_v[pl.ds(k * _L, _L)] = jnp.full((_L,), _DUMP, jnp.int32)
        return 0

    lax.fori_loop(0, _SROWS // _L, init_pos, 0)

    def drain(g, buf, sem):
        gstart = tstart + _G * g

        @pl.when(gstart < tend)
        def _():
            pltpu.make_async_copy(
                ut_hbm.at[:, pl.ds(0, _G * 128)], buf, sem).wait()

    def flush():
        @pl.when(cid == 0)
        def _():
            pltpu.async_copy(stage_v, urows_hbm.at[pos_v], sem_sc).wait()

        @pl.when(cid == 1)
        def _():
            pltpu.async_copy(stage_v, irows_hbm.at[pos_v], sem_sc).wait()

    def process(g, buf, carry):
        cstart = gclamp(g)
        gsafe = jnp.minimum(g, _NG - 1)
        end = bin_s[gsafe]

        def member(c):
            ptr, fill = c
            pv = jnp.full((_L,), gsafe * _GCAP + ptr, jnp.int32)
            b = plsc.load_gather(sb_v, [pv])[0]
            idx = plsc.load_gather(sr_v, [pv])
            colv = idx - cstart * 128
            for d in range(_D // _L):
                stage_v[fill, pl.ds(d * _L, _L)] = plsc.load_gather(
                    buf, [lanes + d * _L, colv])
            plsc.store_scatter(pos_v, [jnp.full((_L,), fill, jnp.int32)],
                               jnp.full((_L,), b, jnp.int32), mask=lane0)
            fill = fill + 1

            @pl.when(fill == _SROWS)
            def _():
                flush()

            fill = jnp.where(fill == _SROWS, 0, fill)
            return ptr + 1, fill

        def cond(c):
            return c[0] < end

        return lax.while_loop(cond, member, (jnp.int32(0), carry))[1]

    def triple_body(gp, carry):
        g0 = 3 * gp
        for j in range(3):
            drain(g0 + j, bufs[j], sems[j])
            carry = process(g0 + j, bufs[j], carry)
            fire(g0 + j + 3, bufs[j], sems[j])
        return carry

    lax.fori_loop(0, (_NG + 2) // 3, triple_body, jnp.int32(0))
    flush()

    @pl.when(cid == 0)
    def _():
        pltpu.make_async_copy(ub_hbm.at[bidx_v], bval_v, sem_b).wait()
        pltpu.sync_copy(bval_v, ubg_hbm.at[pl.ds(bbase, _BPT)])

    @pl.when(cid == 1)
    def _():
        pltpu.make_async_copy(ib_hbm.at[bidx_v], bval_v, sem_b).wait()
        pltpu.sync_copy(bval_v, ibg_hbm.at[pl.ds(bbase, _BPT)])


@functools.partial(
    pl.pallas_call,
    out_shape=jax.ShapeDtypeStruct((_B,), jnp.float32),
    grid=(_B // 2048,),
    in_specs=[
        pl.BlockSpec((2048, 128), lambda g: (g, 0)),
        pl.BlockSpec((2048, 128), lambda g: (g, 0)),
        pl.BlockSpec((2048,), lambda g: (g,)),
        pl.BlockSpec((2048,), lambda g: (g,)),
    ],
    out_specs=pl.BlockSpec((2048,), lambda g: (g,)),
)
def _dot_phase(u_ref, i_ref, ub_ref, ib_ref, o_ref):
    prod = u_ref[:, :_D] * i_ref[:, :_D]
    o_ref[:] = jnp.sum(prod, axis=1) + ub_ref[:] + ib_ref[:]


def kernel(x, u_emb, i_emb, u_bias, i_bias):
    x_flat = x.astype(jnp.int32).reshape(-1)
    urows, irows, ubg, ibg = _gather_phase(
        x_flat, u_emb.T, i_emb.T, u_bias.reshape(-1), i_bias.reshape(-1))
    return _dot_phase(urows, irows, ubg, ibg)


# speedup vs baseline: 1.0100x; 1.0100x over previous
import functools

import jax
import jax.numpy as jnp
from jax import lax
from jax.experimental import pallas as pl
from jax.experimental.pallas import tpu as pltpu
from jax.experimental.pallas import tpu_sc as plsc

_B = 16384
_D = 64
_NC = 2
_NS = 16
_L = 16
_NBLK = 7813
_SPT = 489
_G = 4
_NG = (_SPT + _G - 1) // _G
_CAP = 1536
_GCAP = 40
_SROWS = 64
_OUTROWS = _B + _SROWS
_DUMP = _B
_BPT = _B // _NS
_XCH = 2048

_mesh = plsc.VectorSubcoreMesh(core_axis_name="c", subcore_axis_name="s",
                               num_cores=_NC, num_subcores=_NS)


@functools.partial(
    pl.kernel,
    out_type=(
        jax.ShapeDtypeStruct((_OUTROWS, 128), jnp.float32),
        jax.ShapeDtypeStruct((_OUTROWS, 128), jnp.float32),
        jax.ShapeDtypeStruct((_B,), jnp.float32),
        jax.ShapeDtypeStruct((_B,), jnp.float32),
    ),
    mesh=_mesh,
    compiler_params=pltpu.CompilerParams(
        needs_layout_passes=False,
        use_tc_tiling_on_sc=True,
        disable_bounds_checks=True,
    ),
    scratch_types=[
        pltpu.VMEM((2 * _XCH,), jnp.int32),
        pltpu.VMEM((2 * _XCH,), jnp.int32),
        pltpu.VMEM((_CAP,), jnp.int32),
        pltpu.VMEM((_CAP,), jnp.int32),
        pltpu.VMEM((_NG * _GCAP,), jnp.int32),
        pltpu.VMEM((_NG * _GCAP,), jnp.int32),
        pltpu.VMEM((_D, _G * 128), jnp.float32),
        pltpu.VMEM((_D, _G * 128), jnp.float32),
        pltpu.VMEM((_D, _G * 128), jnp.float32),
        pltpu.VMEM((_SROWS, 128), jnp.float32),
        pltpu.VMEM((_SROWS,), jnp.int32),
        pltpu.VMEM((_BPT,), jnp.int32),
        pltpu.VMEM((_BPT,), jnp.float32),
        pltpu.SMEM((_SPT + 1,), jnp.int32),
        pltpu.SMEM((_SPT + 1,), jnp.int32),
        pltpu.SemaphoreType.DMA,
        pltpu.SemaphoreType.DMA,
        pltpu.SemaphoreType.DMA,
        pltpu.SemaphoreType.DMA,
        pltpu.SemaphoreType.DMA,
        pltpu.SemaphoreType.DMA,
    ],
)
def _gather_phase(x_hbm, ut_hbm, it_hbm, ub_hbm, ib_hbm,
                  urows_hbm, irows_hbm, ubg_hbm, ibg_hbm,
                  xc_v, xd_v, fb_v, fr_v, sb_v, sr_v,
                  gbuf0_v, gbuf1_v, gbuf2_v,
                  stage_v, pos_v, bidx_v, bval_v,
                  bin_s, end_s, sem0, sem1, sem2, sem_sc, sem_b, sem_x):
    cid = lax.axis_index("c")
    sid = lax.axis_index("s")
    lanes = lax.iota(jnp.int32, _L)
    lane0 = lanes == 0
    tau = cid

    tstart = sid * _SPT
    tend = jnp.minimum(tstart + _SPT, _NBLK)
    nbins = tend - tstart

    bbase = sid * _BPT
    pltpu.sync_copy(x_hbm.at[pl.ds(2 * bbase, 2 * _BPT)],
                    xc_v.at[pl.ds(0, 2 * _BPT)])

    def bias_idx_body(k, _):
        bidx_v[pl.ds(k * _L, _L)] = plsc.load_gather(
            xc_v, [2 * (lanes + k * _L) + tau])
        return 0

    lax.fori_loop(0, _BPT // _L, bias_idx_body, 0, unroll=4)

    bufs = (gbuf0_v, gbuf1_v, gbuf2_v)
    sems = (sem0, sem1, sem2)

    def gclamp(g):
        gstart = tstart + _G * g
        return jnp.minimum(gstart, _NBLK - _G)

    def fire(g, buf, sem):
        gstart = tstart + _G * g
        off = pl.multiple_of(gclamp(g) * 128, 128)

        @pl.when((gstart < tend) & (cid == 0))
        def _():
            pltpu.async_copy(ut_hbm.at[:, pl.ds(off, _G * 128)], buf, sem)

        @pl.when((gstart < tend) & (cid == 1))
        def _():
            pltpu.async_copy(it_hbm.at[:, pl.ds(off, _G * 128)], buf, sem)

    fire(0, bufs[0], sems[0])
    fire(1, bufs[1], sems[1])
    fire(2, bufs[2], sems[2])

    @pl.when(cid == 0)
    def _():
        pltpu.async_copy(ub_hbm.at[bidx_v], bval_v, sem_b)

    @pl.when(cid == 1)
    def _():
        pltpu.async_copy(ib_hbm.at[bidx_v], bval_v, sem_b)

    xbufs = (xc_v, xd_v)
    nch = _B // _XCH
    pltpu.async_copy(x_hbm.at[pl.ds(0, 2 * _XCH)], xc_v, sem_x)
    cnt = jnp.int32(0)
    for c8 in range(nch):
        xbuf = xbufs[c8 % 2]
        pltpu.make_async_copy(x_hbm.at[pl.ds(0, 2 * _XCH)], xbuf,
                              sem_x).wait()
        if c8 + 1 < nch:
            pltpu.async_copy(
                x_hbm.at[pl.ds((c8 + 1) * 2 * _XCH, 2 * _XCH)],
                xbufs[(c8 + 1) % 2], sem_x)

        def filt_body(k, cnt, c8=c8, xbuf=xbuf):
            idx = plsc.load_gather(xbuf, [2 * (lanes + k * _L) + tau])
            slab = lax.shift_right_logical(idx, 7)
            m = (slab >= tstart) & (slab < tstart + _SPT)
            rank = plsc.cumsum(m.astype(jnp.int32))
            dest = cnt + rank - 1
            b = c8 * _XCH + k * _L + lanes
            plsc.store_scatter(fb_v, [dest], b, mask=m)
            plsc.store_scatter(fr_v, [dest], idx, mask=m)
            return cnt + plsc.all_reduce_population_count(m)[0]

        cnt = lax.fori_loop(0, _XCH // _L, filt_body, cnt, unroll=4)

    def zero_body(s, _):
        bin_s[s] = 0
        return 0

    lax.fori_loop(0, _NG + 1, zero_body, 0)

    def place_body(e, _):
        ev = jnp.full((_L,), e, jnp.int32)
        idx = plsc.load_gather(fr_v, [ev])
        b = plsc.load_gather(fb_v, [ev])
        g = lax.shift_right_logical(
            lax.shift_right_logical(idx, 7)[0] - tstart, 2)
        o = jnp.minimum(bin_s[g], _GCAP - 1)
        bin_s[g] = o + 1
        ov = jnp.full((_L,), g * _GCAP + o, jnp.int32)
        plsc.store_scatter(sb_v, [ov], b, mask=lane0)
        plsc.store_scatter(sr_v, [ov], idx, mask=lane0)
        return 0

    lax.fori_loop(0, cnt, place_body, 0)

    def init_pos(k, _):
        pos_v[pl.ds(k * _L, _L)] = jnp.full((_L,), _DUMP, jnp.int32)
        return 0

    lax.fori_loop(0, _SROWS // _L, init_pos, 0)

    def drain(g, buf, sem):
        gstart = tstart + _G * g

        @pl.when(gstart < tend)
        def _():
            pltpu.make_async_copy(
                ut_hbm.at[:, pl.ds(0, _G * 128)], buf, sem).wait()

    def flush():
        @pl.when(cid == 0)
        def _():
            pltpu.async_copy(stage_v, urows_hbm.at[pos_v], sem_sc).wait()

        @pl.when(cid == 1)
        def _():
            pltpu.async_copy(stage_v, irows_hbm.at[pos_v], sem_sc).wait()

    def process(g, buf, carry):
        cstart = gclamp(g)
        gsafe = jnp.minimum(g, _NG - 1)
        end = bin_s[gsafe]

        def member(c):
            ptr, fill = c
            pv = jnp.full((_L,), gsafe * _GCAP + ptr, jnp.int32)
            b = plsc.load_gather(sb_v, [pv])[0]
            idx = plsc.load_gather(sr_v, [pv])
            colv = idx - cstart * 128
            for d in range(_D // _L):
                stage_v[fill, pl.ds(d * _L, _L)] = plsc.load_gather(
                    buf, [lanes + d * _L, colv])
            plsc.store_scatter(pos_v, [jnp.full((_L,), fill, jnp.int32)],
                               jnp.full((_L,), b, jnp.int32), mask=lane0)
            fill = fill + 1

            @pl.when(fill == _SROWS)
            def _():
                flush()

            fill = jnp.where(fill == _SROWS, 0, fill)
            return ptr + 1, fill

        def cond(c):
            return c[0] < end

        return lax.while_loop(cond, member, (jnp.int32(0), carry))[1]

    def triple_body(gp, carry):
        g0 = 3 * gp
        for j in range(3):
            drain(g0 + j, bufs[j], sems[j])
            carry = process(g0 + j, bufs[j], carry)
            fire(g0 + j + 3, bufs[j], sems[j])
        return carry

    lax.fori_loop(0, (_NG + 2) // 3, triple_body, jnp.int32(0))
    flush()

    @pl.when(cid == 0)
    def _():
        pltpu.make_async_copy(ub_hbm.at[bidx_v], bval_v, sem_b).wait()
        pltpu.sync_copy(bval_v, ubg_hbm.at[pl.ds(bbase, _BPT)])

    @pl.when(cid == 1)
    def _():
        pltpu.make_async_copy(ib_hbm.at[bidx_v], bval_v, sem_b).wait()
        pltpu.sync_copy(bval_v, ibg_hbm.at[pl.ds(bbase, _BPT)])


@functools.partial(
    pl.pallas_call,
    out_shape=jax.ShapeDtypeStruct((_B,), jnp.float32),
    grid=(_B // 2048,),
    in_specs=[
        pl.BlockSpec((2048, 128), lambda g: (g, 0)),
        pl.BlockSpec((2048, 128), lambda g: (g, 0)),
        pl.BlockSpec((2048,), lambda g: (g,)),
        pl.BlockSpec((2048,), lambda g: (g,)),
    ],
    out_specs=pl.BlockSpec((2048,), lambda g: (g,)),
)
def _dot_phase(u_ref, i_ref, ub_ref, ib_ref, o_ref):
    prod = u_ref[:, :_D] * i_ref[:, :_D]
    o_ref[:] = jnp.sum(prod, axis=1) + ub_ref[:] + ib_ref[:]


def kernel(x, u_emb, i_emb, u_bias, i_bias):
    x_flat = x.astype(jnp.int32).reshape(-1)
    urows, irows, ubg, ibg = _gather_phase(
        x_flat, u_emb.T, i_emb.T, u_bias.reshape(-1), i_bias.reshape(-1))
    return _dot_phase(urows, irows, ubg, ibg)
